# Initial kernel scaffold; baseline (speedup 1.0000x reference)
import jax
import jax.numpy as jnp
from jax.experimental import pallas as pl


def kernel(input_feature, edge_index, edge_values, bias):
    raise NotImplementedError("write your pallas kernel here")


# speedup vs baseline: 3.2459x; 3.2459x over previous
import functools

import jax
import jax.numpy as jnp
from jax import lax
from jax.experimental import pallas as pl
from jax.experimental.pallas import tpu as pltpu
from jax.experimental.pallas import tpu_sc as plsc

NC = 2
NS = 16
L = 16
CHUNK = 128
N_PAD = 10240
ROWS_PER_TILE = N_PAD // NS
INIT_ROWS = 128


def _sc_kernel(e_per_tile, half, xstack, srcs, dst, vals, bias_h, out, acc,
               src_v, dst_v, vals_v, rows_v, binit_v, bias_v, sem):
    cid = lax.axis_index("c")
    sid = lax.axis_index("s")
    e_pad = e_per_tile * NS
    n_chunks = e_per_tile // CHUNK

    pltpu.sync_copy(bias_h, bias_v)

    def _binit_row(i, _):
        for f in range(half // L):
            binit_v[i, pl.ds(f * L, L)] = bias_v[pl.ds(cid * half + f * L, L)]
        return 0

    lax.fori_loop(0, INIT_ROWS, _binit_row, 0, unroll=False)

    for k in range(ROWS_PER_TILE // INIT_ROWS):
        pltpu.sync_copy(
            binit_v,
            acc.at[pl.ds(sid * ROWS_PER_TILE + k * INIT_ROWS, INIT_ROWS)])
    plsc.subcore_barrier()

    def _chunk(ch, _):
        off = sid * e_per_tile + ch * CHUNK
        pltpu.sync_copy(srcs.at[pl.ds(cid * e_pad + off, CHUNK)], src_v)
        pltpu.sync_copy(dst.at[pl.ds(off, CHUNK)], dst_v)
        pltpu.sync_copy(vals.at[pl.ds(off, CHUNK)], vals_v)
        pltpu.async_copy(xstack.at[src_v], rows_v, sem).wait()

        def _scale_group(g, _):
            vv = vals_v[pl.ds(g * L, L)]
            for k in range(L):
                i = g * L + k
                vk = vv[k]
                for f in range(half // L):
                    sl = pl.ds(f * L, L)
                    rows_v[i, sl] = rows_v[i, sl] * vk
            return 0

        lax.fori_loop(0, CHUNK // L, _scale_group, 0, unroll=False)
        pltpu.sync_copy(rows_v, acc.at[dst_v], add=True)
        return 0

    lax.fori_loop(0, n_chunks, _chunk, 0, unroll=False)
    plsc.subcore_barrier()

    pltpu.sync_copy(
        acc.at[pl.ds(sid * ROWS_PER_TILE, ROWS_PER_TILE)],
        out.at[cid, pl.ds(sid * ROWS_PER_TILE, ROWS_PER_TILE)])


def kernel(input_feature, edge_index, edge_values, bias):
    n, d = input_feature.shape
    half = d // NC
    e = edge_index.shape[1]
    e_per_tile = -(-e // (NS * CHUNK)) * CHUNK
    e_pad = e_per_tile * NS
    src = edge_index[0].astype(jnp.int32)
    dst = edge_index[1].astype(jnp.int32)
    vals = edge_values.astype(jnp.float32)
    if e_pad != e:
        src = jnp.pad(src, (0, e_pad - e))
        dst = jnp.pad(dst, (0, e_pad - e))
        vals = jnp.pad(vals, (0, e_pad - e))
    xstack = input_feature.reshape(n, NC, half).transpose(1, 0, 2).reshape(
        NC * n, half)
    srcs = jnp.concatenate([src, src + n])

    mesh = plsc.VectorSubcoreMesh(
        core_axis_name="c", subcore_axis_name="s", num_cores=NC,
        num_subcores=NS)
    out = pl.kernel(
        functools.partial(_sc_kernel, e_per_tile, half),
        out_type=jax.ShapeDtypeStruct((NC, N_PAD, half), jnp.float32),
        mesh=mesh,
        scratch_types=[
            pltpu.VMEM_SHARED((N_PAD, half), jnp.float32),
            pltpu.VMEM((CHUNK,), jnp.int32),
            pltpu.VMEM((CHUNK,), jnp.int32),
            pltpu.VMEM((CHUNK,), jnp.float32),
            pltpu.VMEM((CHUNK, half), jnp.float32),
            pltpu.VMEM((INIT_ROWS, half), jnp.float32),
            pltpu.VMEM((d,), jnp.float32),
            pltpu.SemaphoreType.DMA,
        ],
        compiler_params=pltpu.CompilerParams(use_tc_tiling_on_sc=False),
    )(xstack, srcs, dst, vals, bias)
    return out[:, :n].transpose(1, 0, 2).reshape(n, d)


# speedup vs baseline: 4.2282x; 1.3026x over previous
import functools

import jax
import jax.numpy as jnp
from jax import lax
from jax.experimental import pallas as pl
from jax.experimental.pallas import tpu as pltpu
from jax.experimental.pallas import tpu_sc as plsc

NC = 2
NS = 16
L = 16
CHUNK = 128
G = 4
N_PAD = 10240
ROWS_PER_TILE = N_PAD // NS
INIT_ROWS = 128
M_SRC, M_DST, M_VAL = 0, 1, 2


def _sc_kernel(e_per_tile, half, xstack, meta, bias_h, out, acc,
               meta_v, rows_v, binit_v, bias_v, sem_idx, sem_g, sem_s):
    cid = lax.axis_index("c")
    sid = lax.axis_index("s")
    n_chunks_tile = e_per_tile // CHUNK
    n_pairs = n_chunks_tile // (2 * G)
    chunk0 = sid * n_chunks_tile

    def idx_start(pset, b, chunk_id):
        pltpu.async_copy(meta.at[cid, chunk_id], meta_v.at[pset, b],
                         sem_idx.at[pset, b])

    def idx_wait(pset, b):
        pltpu.make_async_copy(meta.at[0, 0], meta_v.at[pset, b],
                              sem_idx.at[pset, b]).wait()

    def gather_start(pset, b):
        pltpu.async_copy(xstack.at[meta_v.at[pset, b, M_SRC]], rows_v.at[b],
                         sem_g.at[b])

    def gather_wait(b):
        pltpu.make_async_copy(xstack.at[pl.ds(0, CHUNK)], rows_v.at[b],
                              sem_g.at[b]).wait()

    def scatter_start(pset, b):
        pltpu.async_copy(rows_v.at[b], acc.at[meta_v.at[pset, b, M_DST]],
                         sem_s.at[b], add=True)

    def scatter_wait(b):
        pltpu.make_async_copy(rows_v.at[b], acc.at[pl.ds(0, CHUNK)],
                              sem_s.at[b]).wait()

    def scale(pset, b):
        def _grp(g, _):
            vv = plsc.bitcast(meta_v[pset, b, M_VAL, pl.ds(g * L, L)],
                              jnp.float32)
            for k in range(L):
                i = g * L + k
                vk = vv[k]
                for f in range(half // L):
                    sl = pl.ds(f * L, L)
                    rows_v[b, i, sl] = rows_v[b, i, sl] * vk
            return 0

        lax.fori_loop(0, CHUNK // L, _grp, 0, unroll=False)

    pltpu.sync_copy(bias_h, bias_v)

    def _binit_row(i, _):
        for f in range(half // L):
            binit_v[i, pl.ds(f * L, L)] = bias_v[pl.ds(cid * half + f * L, L)]
        return 0

    lax.fori_loop(0, INIT_ROWS, _binit_row, 0, unroll=False)

    for k in range(ROWS_PER_TILE // INIT_ROWS):
        pltpu.sync_copy(
            binit_v,
            acc.at[pl.ds(sid * ROWS_PER_TILE + k * INIT_ROWS, INIT_ROWS)])
    plsc.subcore_barrier()

    for b in range(G):
        idx_start(0, b, chunk0 + b)

    def _pair(j, _):
        for phase in range(2):
            myset, nxtset = phase, 1 - phase
            g = 2 * j + phase
            if phase == 0:
                @pl.when(j > 0)
                def _():
                    for b in range(G):
                        scatter_wait(b)
            else:
                for b in range(G):
                    scatter_wait(b)
            for b in range(G):
                idx_wait(myset, b)
                gather_start(myset, b)
            if phase == 0:
                for b in range(G):
                    idx_start(nxtset, b, chunk0 + (g + 1) * G + b)
            else:
                @pl.when(j + 1 < n_pairs)
                def _():
                    for b in range(G):
                        idx_start(nxtset, b, chunk0 + (g + 1) * G + b)
            for b in range(G):
                gather_wait(b)
                scale(myset, b)
                scatter_start(myset, b)
        return 0

    lax.fori_loop(0, n_pairs, _pair, 0, unroll=False)
    for b in range(G):
        scatter_wait(b)
    plsc.subcore_barrier()

    pltpu.sync_copy(
        acc.at[pl.ds(sid * ROWS_PER_TILE, ROWS_PER_TILE)],
        out.at[cid, pl.ds(sid * ROWS_PER_TILE, ROWS_PER_TILE)])


def kernel(input_feature, edge_index, edge_values, bias):
    n, d = input_feature.shape
    half = d // NC
    e = edge_index.shape[1]
    quantum = CHUNK * 2 * G
    e_per_tile = -(-e // (NS * quantum)) * quantum
    e_pad = e_per_tile * NS
    src = edge_index[0].astype(jnp.int32)
    dst = edge_index[1].astype(jnp.int32)
    vals = edge_values.astype(jnp.float32)
    if e_pad != e:
        src = jnp.pad(src, (0, e_pad - e))
        dst = jnp.pad(dst, (0, e_pad - e))
        vals = jnp.pad(vals, (0, e_pad - e))
    xstack = input_feature.reshape(n, NC, half).transpose(1, 0, 2).reshape(
        NC * n, half)
    src_r = src.reshape(-1, CHUNK)
    dst_r = dst.reshape(-1, CHUNK)
    val_r = lax.bitcast_convert_type(vals, jnp.int32).reshape(-1, CHUNK)
    meta = jnp.stack([
        jnp.stack([src_r, dst_r, val_r], axis=1),
        jnp.stack([src_r + n, dst_r, val_r], axis=1),
    ])

    mesh = plsc.VectorSubcoreMesh(
        core_axis_name="c", subcore_axis_name="s", num_cores=NC,
        num_subcores=NS)
    out = pl.kernel(
        functools.partial(_sc_kernel, e_per_tile, half),
        out_type=jax.ShapeDtypeStruct((NC, N_PAD, half), jnp.float32),
        mesh=mesh,
        scratch_types=[
            pltpu.VMEM_SHARED((N_PAD, half), jnp.float32),
            pltpu.VMEM((2, G, 3, CHUNK), jnp.int32),
            pltpu.VMEM((G, CHUNK, half), jnp.float32),
            pltpu.VMEM((INIT_ROWS, half), jnp.float32),
            pltpu.VMEM((d,), jnp.float32),
            pltpu.SemaphoreType.DMA((2, G)),
            pltpu.SemaphoreType.DMA((G,)),
            pltpu.SemaphoreType.DMA((G,)),
        ],
        compiler_params=pltpu.CompilerParams(use_tc_tiling_on_sc=False,
                                             needs_layout_passes=False),
    )(xstack, meta, bias)
    return out[:, :n].transpose(1, 0, 2).reshape(n, d)


# speedup vs baseline: 7.2973x; 1.7259x over previous
import functools

import jax
import jax.numpy as jnp
from jax import lax
from jax.experimental import pallas as pl
from jax.experimental.pallas import tpu as pltpu
from jax.experimental.pallas import tpu_sc as plsc

NC = 2
NS = 16
L = 16
CHUNK = 128
G = 4
N_PAD = 10240
ROWS_PER_TILE = N_PAD // NS
INIT_ROWS = 128
M_SRC, M_DST, M_VAL = 0, 1, 2


def _sc_kernel(e_per_tile, half, xh, meta, bias_h, out, acc, xsh,
               meta_v, rows_v, binit_v, bias_v, sem_idx, sem_g, sem_s):
    cid = lax.axis_index("c")
    sid = lax.axis_index("s")
    n_chunks_tile = e_per_tile // CHUNK
    n_pairs = n_chunks_tile // (2 * G)
    chunk0 = sid * n_chunks_tile

    def idx_start(pset, b, chunk_id):
        pltpu.async_copy(meta.at[chunk_id], meta_v.at[pset, b],
                         sem_idx.at[pset, b])

    def idx_wait(pset, b):
        pltpu.make_async_copy(meta.at[0], meta_v.at[pset, b],
                              sem_idx.at[pset, b]).wait()

    def gather_start(pset, b):
        pltpu.async_copy(xsh.at[meta_v.at[pset, b, M_SRC]], rows_v.at[b],
                         sem_g.at[b])

    def gather_wait(b):
        pltpu.make_async_copy(xh.at[0, pl.ds(0, CHUNK)], rows_v.at[b],
                              sem_g.at[b]).wait()

    def scatter_start(pset, b):
        pltpu.async_copy(rows_v.at[b], acc.at[meta_v.at[pset, b, M_DST]],
                         sem_s.at[b], add=True)

    def scatter_wait(b):
        pltpu.make_async_copy(rows_v.at[b], acc.at[pl.ds(0, CHUNK)],
                              sem_s.at[b]).wait()

    def scale(pset, b):
        def _grp(g, _):
            vv = plsc.bitcast(meta_v[pset, b, M_VAL, pl.ds(g * L, L)],
                              jnp.float32)
            for k in range(L):
                i = g * L + k
                vk = vv[k]
                for f in range(half // L):
                    sl = pl.ds(f * L, L)
                    rows_v[b, i, sl] = rows_v[b, i, sl] * vk
            return 0

        lax.fori_loop(0, CHUNK // L, _grp, 0, unroll=False)

    pltpu.sync_copy(xh.at[cid, pl.ds(sid * ROWS_PER_TILE, ROWS_PER_TILE)],
                    xsh.at[pl.ds(sid * ROWS_PER_TILE, ROWS_PER_TILE)])

    pltpu.sync_copy(bias_h, bias_v)

    def _binit_row(i, _):
        for f in range(half // L):
            binit_v[i, pl.ds(f * L, L)] = bias_v[pl.ds(cid * half + f * L, L)]
        return 0

    lax.fori_loop(0, INIT_ROWS, _binit_row, 0, unroll=False)

    for k in range(ROWS_PER_TILE // INIT_ROWS):
        pltpu.sync_copy(
            binit_v,
            acc.at[pl.ds(sid * ROWS_PER_TILE + k * INIT_ROWS, INIT_ROWS)])
    plsc.subcore_barrier()

    for b in range(G):
        idx_start(0, b, chunk0 + b)

    def _pair(j, _):
        for phase in range(2):
            myset, nxtset = phase, 1 - phase
            g = 2 * j + phase
            if phase == 0:
                @pl.when(j > 0)
                def _():
                    for b in range(G):
                        scatter_wait(b)
            else:
                for b in range(G):
                    scatter_wait(b)
            for b in range(G):
                idx_wait(myset, b)
                gather_start(myset, b)
            if phase == 0:
                for b in range(G):
                    idx_start(nxtset, b, chunk0 + (g + 1) * G + b)
            else:
                @pl.when(j + 1 < n_pairs)
                def _():
                    for b in range(G):
                        idx_start(nxtset, b, chunk0 + (g + 1) * G + b)
            for b in range(G):
                gather_wait(b)
                scale(myset, b)
                scatter_start(myset, b)
        return 0

    lax.fori_loop(0, n_pairs, _pair, 0, unroll=False)
    for b in range(G):
        scatter_wait(b)
    plsc.subcore_barrier()

    pltpu.sync_copy(
        acc.at[pl.ds(sid * ROWS_PER_TILE, ROWS_PER_TILE)],
        out.at[cid, pl.ds(sid * ROWS_PER_TILE, ROWS_PER_TILE)])


def kernel(input_feature, edge_index, edge_values, bias):
    n, d = input_feature.shape
    half = d // NC
    e = edge_index.shape[1]
    quantum = CHUNK * 2 * G
    e_per_tile = -(-e // (NS * quantum)) * quantum
    e_pad = e_per_tile * NS
    src = edge_index[0].astype(jnp.int32)
    dst = edge_index[1].astype(jnp.int32)
    vals = edge_values.astype(jnp.float32)
    if e_pad != e:
        src = jnp.pad(src, (0, e_pad - e))
        dst = jnp.pad(dst, (0, e_pad - e))
        vals = jnp.pad(vals, (0, e_pad - e))
    xh = jnp.pad(input_feature, ((0, N_PAD - n), (0, 0))).reshape(
        N_PAD, NC, half).transpose(1, 0, 2)
    meta = jnp.stack([
        src.reshape(-1, CHUNK),
        dst.reshape(-1, CHUNK),
        lax.bitcast_convert_type(vals, jnp.int32).reshape(-1, CHUNK),
    ], axis=1)

    mesh = plsc.VectorSubcoreMesh(
        core_axis_name="c", subcore_axis_name="s", num_cores=NC,
        num_subcores=NS)
    out = pl.kernel(
        functools.partial(_sc_kernel, e_per_tile, half),
        out_type=jax.ShapeDtypeStruct((NC, N_PAD, half), jnp.float32),
        mesh=mesh,
        scratch_types=[
            pltpu.VMEM_SHARED((N_PAD, half), jnp.float32),
            pltpu.VMEM_SHARED((N_PAD, half), jnp.float32),
            pltpu.VMEM((2, G, 3, CHUNK), jnp.int32),
            pltpu.VMEM((G, CHUNK, half), jnp.float32),
            pltpu.VMEM((INIT_ROWS, half), jnp.float32),
            pltpu.VMEM((d,), jnp.float32),
            pltpu.SemaphoreType.DMA((2, G)),
            pltpu.SemaphoreType.DMA((G,)),
            pltpu.SemaphoreType.DMA((G,)),
        ],
        compiler_params=pltpu.CompilerParams(use_tc_tiling_on_sc=False,
                                             needs_layout_passes=False),
    )(xh, meta, bias)
    return out[:, :n].transpose(1, 0, 2).reshape(n, d)


# speedup vs baseline: 8.8993x; 1.2195x over previous
import functools

import jax
import jax.numpy as jnp
from jax import lax
from jax.experimental import pallas as pl
from jax.experimental.pallas import tpu as pltpu
from jax.experimental.pallas import tpu_sc as plsc

NC = 2
NS = 16
L = 16
CHUNK = 128
G = 4
INIT_ROWS = 125
M_SRC, M_DST, M_VAL = 0, 1, 2


def _sc_kernel(e_per_tile, half, n, x, meta, bias_h, out, acc, xsh,
               meta_v, rows_v, binit_v, bias_v, sem_idx, sem_g, sem_s):
    cid = lax.axis_index("c")
    sid = lax.axis_index("s")
    n_chunks_tile = e_per_tile // CHUNK
    n_pairs = n_chunks_tile // (2 * G)
    chunk0 = sid * n_chunks_tile
    rpt = n // NS

    def idx_start(pset, b, chunk_id):
        pltpu.async_copy(meta.at[chunk_id], meta_v.at[pset, b],
                         sem_idx.at[pset, b])

    def idx_wait(pset, b):
        pltpu.make_async_copy(meta.at[0], meta_v.at[pset, b],
                              sem_idx.at[pset, b]).wait()

    def gather_start(pset, b):
        pltpu.async_copy(xsh.at[meta_v.at[pset, b, M_SRC]], rows_v.at[b],
                         sem_g.at[b])

    def gather_wait(b):
        pltpu.make_async_copy(x.at[pl.ds(0, CHUNK), pl.ds(0, half)],
                              rows_v.at[b], sem_g.at[b]).wait()

    def scatter_start(pset, b):
        pltpu.async_copy(rows_v.at[b], acc.at[meta_v.at[pset, b, M_DST]],
                         sem_s.at[b], add=True)

    def scatter_wait(b):
        pltpu.make_async_copy(rows_v.at[b], acc.at[pl.ds(0, CHUNK)],
                              sem_s.at[b]).wait()

    def scale(pset, b):
        def _grp(g, _):
            vv = plsc.bitcast(meta_v[pset, b, M_VAL, pl.ds(g * L, L)],
                              jnp.float32)
            for k in range(L):
                i = g * L + k
                vk = vv[k]
                for f in range(half // L):
                    sl = pl.ds(f * L, L)
                    rows_v[b, i, sl] = rows_v[b, i, sl] * vk
            return 0

        lax.fori_loop(0, CHUNK // L, _grp, 0, unroll=False)

    rows = pl.ds(sid * rpt, rpt)
    for c in range(NC):
        @pl.when(cid == c)
        def _():
            pltpu.sync_copy(x.at[rows, pl.ds(c * half, half)], xsh.at[rows])

    pltpu.sync_copy(bias_h, bias_v)

    def _binit_row(i, _):
        for f in range(half // L):
            binit_v[i, pl.ds(f * L, L)] = bias_v[pl.ds(cid * half + f * L, L)]
        return 0

    lax.fori_loop(0, INIT_ROWS, _binit_row, 0, unroll=False)

    for k in range(rpt // INIT_ROWS):
        pltpu.sync_copy(
            binit_v, acc.at[pl.ds(sid * rpt + k * INIT_ROWS, INIT_ROWS)])
    plsc.subcore_barrier()

    for b in range(G):
        idx_start(0, b, chunk0 + b)

    def _pair(j, _):
        for phase in range(2):
            myset, nxtset = phase, 1 - phase
            g = 2 * j + phase
            if phase == 0:
                @pl.when(j > 0)
                def _():
                    for b in range(G):
                        scatter_wait(b)
            else:
                for b in range(G):
                    scatter_wait(b)
            for b in range(G):
                idx_wait(myset, b)
                gather_start(myset, b)
            if phase == 0:
                for b in range(G):
                    idx_start(nxtset, b, chunk0 + (g + 1) * G + b)
            else:
                @pl.when(j + 1 < n_pairs)
                def _():
                    for b in range(G):
                        idx_start(nxtset, b, chunk0 + (g + 1) * G + b)
            for b in range(G):
                gather_wait(b)
                scale(myset, b)
                scatter_start(myset, b)
        return 0

    lax.fori_loop(0, n_pairs, _pair, 0, unroll=False)
    for b in range(G):
        scatter_wait(b)
    plsc.subcore_barrier()

    for c in range(NC):
        @pl.when(cid == c)
        def _():
            pltpu.sync_copy(acc.at[rows], out.at[rows, pl.ds(c * half, half)])


def kernel(input_feature, edge_index, edge_values, bias):
    n, d = input_feature.shape
    half = d // NC
    e = edge_index.shape[1]
    quantum = CHUNK * 2 * G
    e_per_tile = -(-e // (NS * quantum)) * quantum
    e_pad = e_per_tile * NS
    src = edge_index[0].astype(jnp.int32)
    dst = edge_index[1].astype(jnp.int32)
    vals = edge_values.astype(jnp.float32)
    if e_pad != e:
        src = jnp.pad(src, (0, e_pad - e))
        dst = jnp.pad(dst, (0, e_pad - e))
        vals = jnp.pad(vals, (0, e_pad - e))
    meta = jnp.stack([
        src.reshape(-1, CHUNK),
        dst.reshape(-1, CHUNK),
        lax.bitcast_convert_type(vals, jnp.int32).reshape(-1, CHUNK),
    ], axis=1)

    mesh = plsc.VectorSubcoreMesh(
        core_axis_name="c", subcore_axis_name="s", num_cores=NC,
        num_subcores=NS)
    return pl.kernel(
        functools.partial(_sc_kernel, e_per_tile, half, n),
        out_type=jax.ShapeDtypeStruct((n, d), jnp.float32),
        mesh=mesh,
        scratch_types=[
            pltpu.VMEM_SHARED((n, half), jnp.float32),
            pltpu.VMEM_SHARED((n, half), jnp.float32),
            pltpu.VMEM((2, G, 3, CHUNK), jnp.int32),
            pltpu.VMEM((G, CHUNK, half), jnp.float32),
            pltpu.VMEM((INIT_ROWS, half), jnp.float32),
            pltpu.VMEM((d,), jnp.float32),
            pltpu.SemaphoreType.DMA((2, G)),
            pltpu.SemaphoreType.DMA((G,)),
            pltpu.SemaphoreType.DMA((G,)),
        ],
        compiler_params=pltpu.CompilerParams(use_tc_tiling_on_sc=False,
                                             needs_layout_passes=False),
    )(input_feature, meta, bias)


# speedup vs baseline: 9.0517x; 1.0171x over previous
import functools

import jax
import jax.numpy as jnp
from jax import lax
from jax.experimental import pallas as pl
from jax.experimental.pallas import tpu as pltpu
from jax.experimental.pallas import tpu_sc as plsc

NC = 2
NS = 16
L = 16
CHUNK = 128
G = 4
INIT_ROWS = 125


def _sc_kernel(e_per_tile, half, n, x, src, dst, vals, bias_h, out, acc, xsh,
               src_v, dst_v, vals_v, rows_v, binit_v, bias_v,
               sem_idx, sem_g, sem_s):
    cid = lax.axis_index("c")
    sid = lax.axis_index("s")
    n_chunks_tile = e_per_tile // CHUNK
    n_pairs = n_chunks_tile // (2 * G)
    chunk0 = sid * n_chunks_tile
    rpt = n // NS

    def idx_start(pset, b, chunk_id):
        off = pl.ds(chunk_id * CHUNK, CHUNK)
        pltpu.async_copy(src.at[off], src_v.at[pset, b], sem_idx.at[pset, b])
        pltpu.async_copy(dst.at[off], dst_v.at[pset, b], sem_idx.at[pset, b])
        pltpu.async_copy(vals.at[off], vals_v.at[pset, b],
                         sem_idx.at[pset, b])

    def idx_wait(pset, b):
        off = pl.ds(0, CHUNK)
        pltpu.make_async_copy(src.at[off], src_v.at[pset, b],
                              sem_idx.at[pset, b]).wait()
        pltpu.make_async_copy(dst.at[off], dst_v.at[pset, b],
                              sem_idx.at[pset, b]).wait()
        pltpu.make_async_copy(vals.at[off], vals_v.at[pset, b],
                              sem_idx.at[pset, b]).wait()

    def gather_start(pset, b):
        pltpu.async_copy(xsh.at[src_v.at[pset, b]], rows_v.at[b],
                         sem_g.at[b])

    def gather_wait(b):
        pltpu.make_async_copy(x.at[pl.ds(0, CHUNK), pl.ds(0, half)],
                              rows_v.at[b], sem_g.at[b]).wait()

    def scatter_start(pset, b):
        pltpu.async_copy(rows_v.at[b], acc.at[dst_v.at[pset, b]],
                         sem_s.at[b], add=True)

    def scatter_wait(b):
        pltpu.make_async_copy(rows_v.at[b], acc.at[pl.ds(0, CHUNK)],
                              sem_s.at[b]).wait()

    def scale(pset, b):
        def _grp(g, _):
            vv = vals_v[pset, b, pl.ds(g * L, L)]
            for k in range(L):
                i = g * L + k
                vk = vv[k]
                for f in range(half // L):
                    sl = pl.ds(f * L, L)
                    rows_v[b, i, sl] = rows_v[b, i, sl] * vk
            return 0

        lax.fori_loop(0, CHUNK // L, _grp, 0, unroll=False)

    rows = pl.ds(sid * rpt, rpt)
    for c in range(NC):
        @pl.when(cid == c)
        def _():
            pltpu.sync_copy(x.at[rows, pl.ds(c * half, half)], xsh.at[rows])

    pltpu.sync_copy(bias_h, bias_v)

    def _binit_row(i, _):
        for f in range(half // L):
            binit_v[i, pl.ds(f * L, L)] = bias_v[pl.ds(cid * half + f * L, L)]
        return 0

    lax.fori_loop(0, INIT_ROWS, _binit_row, 0, unroll=False)

    for k in range(rpt // INIT_ROWS):
        pltpu.sync_copy(
            binit_v, acc.at[pl.ds(sid * rpt + k * INIT_ROWS, INIT_ROWS)])
    plsc.subcore_barrier()

    for b in range(G):
        idx_start(0, b, chunk0 + b)

    def _pair(j, _):
        for phase in range(2):
            myset, nxtset = phase, 1 - phase
            g = 2 * j + phase
            if phase == 0:
                @pl.when(j > 0)
                def _():
                    for b in range(G):
                        scatter_wait(b)
            else:
                for b in range(G):
                    scatter_wait(b)
            for b in range(G):
                idx_wait(myset, b)
                gather_start(myset, b)
            if phase == 0:
                for b in range(G):
                    idx_start(nxtset, b, chunk0 + (g + 1) * G + b)
            else:
                @pl.when(j + 1 < n_pairs)
                def _():
                    for b in range(G):
                        idx_start(nxtset, b, chunk0 + (g + 1) * G + b)
            for b in range(G):
                gather_wait(b)
                scale(myset, b)
                scatter_start(myset, b)
        return 0

    lax.fori_loop(0, n_pairs, _pair, 0, unroll=False)
    for b in range(G):
        scatter_wait(b)
    plsc.subcore_barrier()

    for c in range(NC):
        @pl.when(cid == c)
        def _():
            pltpu.sync_copy(acc.at[rows], out.at[rows, pl.ds(c * half, half)])


def kernel(input_feature, edge_index, edge_values, bias):
    n, d = input_feature.shape
    half = d // NC
    e = edge_index.shape[1]
    quantum = CHUNK * 2 * G
    e_per_tile = -(-e // (NS * quantum)) * quantum
    e_pad = e_per_tile * NS
    src = edge_index[0].astype(jnp.int32)
    dst = edge_index[1].astype(jnp.int32)
    vals = edge_values.astype(jnp.float32)
    if e_pad != e:
        src = jnp.pad(src, (0, e_pad - e))
        dst = jnp.pad(dst, (0, e_pad - e))
        vals = jnp.pad(vals, (0, e_pad - e))
    mesh = plsc.VectorSubcoreMesh(
        core_axis_name="c", subcore_axis_name="s", num_cores=NC,
        num_subcores=NS)
    return pl.kernel(
        functools.partial(_sc_kernel, e_per_tile, half, n),
        out_type=jax.ShapeDtypeStruct((n, d), jnp.float32),
        mesh=mesh,
        scratch_types=[
            pltpu.VMEM_SHARED((n, half), jnp.float32),
            pltpu.VMEM_SHARED((n, half), jnp.float32),
            pltpu.VMEM((2, G, CHUNK), jnp.int32),
            pltpu.VMEM((2, G, CHUNK), jnp.int32),
            pltpu.VMEM((2, G, CHUNK), jnp.float32),
            pltpu.VMEM((G, CHUNK, half), jnp.float32),
            pltpu.VMEM((INIT_ROWS, half), jnp.float32),
            pltpu.VMEM((d,), jnp.float32),
            pltpu.SemaphoreType.DMA((2, G)),
            pltpu.SemaphoreType.DMA((G,)),
            pltpu.SemaphoreType.DMA((G,)),
        ],
        compiler_params=pltpu.CompilerParams(use_tc_tiling_on_sc=False,
                                             needs_layout_passes=False),
    )(input_feature, src, dst, vals, bias)
